# baseline (speedup 1.0000x reference)
import functools

import jax
import jax.numpy as jnp
from jax import lax
from jax.experimental import pallas as pl
from jax.experimental.pallas import tpu as pltpu
from jax.experimental.pallas import tpu_sc as plsc

B = 16384
F = 32
NC = 2
NS = 16
NW = NC * NS
BPW = B // NW
GCHUNK = 128
NCHUNK = BPW // GCHUNK
L = 16


def _gmf_body(users_hbm, items_hbm, utab_hbm, itab_hbm, params_hbm, out_hbm,
              idx_u, idx_i, rows_u, rows_i, params_v, out_v, sem_u, sem_i):
    wid = lax.axis_index("s") * NC + lax.axis_index("c")
    base = wid * BPW

    for j in range(NCHUNK):
        pltpu.sync_copy(users_hbm.at[pl.ds(base + j * GCHUNK, GCHUNK)],
                        idx_u.at[j])
        pltpu.sync_copy(items_hbm.at[pl.ds(base + j * GCHUNK, GCHUNK)],
                        idx_i.at[j])
    pltpu.sync_copy(params_hbm, params_v)

    copies = []
    for j in range(NCHUNK):
        copies.append(pltpu.async_copy(
            utab_hbm.at[idx_u.at[j]],
            rows_u.at[pl.ds(j * GCHUNK, GCHUNK)], sem_u))
        copies.append(pltpu.async_copy(
            itab_hbm.at[idx_i.at[j]],
            rows_i.at[pl.ds(j * GCHUNK, GCHUNK)], sem_i))
    for c in copies:
        c.wait()

    wv = [params_v[pl.ds(k * L, L)] for k in range(F // L)]
    bv = params_v[pl.ds(F // L * L, L)]
    w = [wv[f // L][f % L] for f in range(F)]
    bias = bv[0]
    lane = lax.iota(jnp.int32, L)

    def group(g, carry):
        rows = g * L + lane
        acc = jnp.zeros((L,), jnp.float32)
        for f in range(F):
            cols = jnp.full((L,), f, jnp.int32)
            uv = plsc.load_gather(rows_u, [rows, cols])
            iv = plsc.load_gather(rows_i, [rows, cols])
            acc = acc + (uv * iv) * w[f]
        z = acc + bias
        out_v[pl.ds(g * L, L)] = 1.0 / (1.0 + jnp.exp(-z))
        return carry

    lax.fori_loop(0, BPW // L, group, 0)

    pltpu.sync_copy(out_v, out_hbm.at[pl.ds(base, BPW)])


_gmf = functools.partial(
    pl.kernel,
    out_type=jax.ShapeDtypeStruct((B,), jnp.float32),
    mesh=plsc.VectorSubcoreMesh(core_axis_name="c", subcore_axis_name="s"),
    scratch_types=[
        pltpu.VMEM((NCHUNK, GCHUNK), jnp.int32),
        pltpu.VMEM((NCHUNK, GCHUNK), jnp.int32),
        pltpu.VMEM((BPW, F), jnp.float32),
        pltpu.VMEM((BPW, F), jnp.float32),
        pltpu.VMEM((48,), jnp.float32),
        pltpu.VMEM((BPW,), jnp.float32),
        pltpu.SemaphoreType.DMA,
        pltpu.SemaphoreType.DMA,
    ],
    compiler_params=pltpu.CompilerParams(needs_layout_passes=False,
                                         use_tc_tiling_on_sc=False),
)(_gmf_body)


def kernel(users, items, user_table, item_table, pred_w, pred_b):
    params = jnp.concatenate([
        pred_w.reshape(-1).astype(jnp.float32),
        pred_b.reshape(-1).astype(jnp.float32),
        jnp.zeros((48 - F - 1,), jnp.float32),
    ])
    return _gmf(users.astype(jnp.int32), items.astype(jnp.int32),
                user_table, item_table, params)


# speedup vs baseline: 3.5415x; 3.5415x over previous
import functools

import jax
import jax.numpy as jnp
from jax import lax
from jax.experimental import pallas as pl
from jax.experimental.pallas import tpu as pltpu
from jax.experimental.pallas import tpu_sc as plsc

B = 16384
F = 32
NC = 2
NS = 16
NW = NC * NS
BPW = B // NW
L = 16
TS = 8
TL = 128
HALF = F // 2
NWAVE = BPW // L
PROWS = L * HALF


def _gmf_body(users_hbm, items_hbm, utab_hbm, itab_hbm, params_hbm, out_hbm,
              idx_u, idx_i, pan_u, pan_i, params_v, out_v, sem_u, sem_i):
    wid = lax.axis_index("s") * NC + lax.axis_index("c")
    base = wid * BPW

    pltpu.sync_copy(users_hbm.at[pl.ds(base, BPW)], idx_u)
    pltpu.sync_copy(items_hbm.at[pl.ds(base, BPW)], idx_i)
    pltpu.sync_copy(params_hbm, params_v)

    wv = [params_v[pl.ds(k * L, L)] for k in range(F // L)]
    bv = params_v[pl.ds(F // L * L, L)]
    w = [wv[f // L][f % L] for f in range(F)]
    bias = bv[0]
    lane = lax.iota(jnp.int32, L)

    def wave(v, carry):
        uvec = idx_u[pl.ds(v * L, L)]
        ivec = idx_i[pl.ds(v * L, L)]
        rem_u = uvec - (uvec // TL) * TL
        rem_i = ivec - (ivec // TL) * TL
        acc = jnp.zeros((L,), jnp.float32)
        for half in range(2):
            copies = []
            for k in range(L):
                qu = pl.multiple_of((uvec[k] // TL) * TL, TL)
                qi = pl.multiple_of((ivec[k] // TL) * TL, TL)
                for t in range(2):
                    fr = half * HALF + t * TS
                    copies.append(pltpu.async_copy(
                        utab_hbm.at[pl.ds(fr, TS), pl.ds(qu, TL)],
                        pan_u.at[pl.ds(k * HALF + t * TS, TS)], sem_u))
                    copies.append(pltpu.async_copy(
                        itab_hbm.at[pl.ds(fr, TS), pl.ds(qi, TL)],
                        pan_i.at[pl.ds(k * HALF + t * TS, TS)], sem_i))
            for c in copies:
                c.wait()
            for fo in range(HALF):
                f = half * HALF + fo
                rows = lane * HALF + fo
                ucol = plsc.load_gather(pan_u, [rows, rem_u])
                icol = plsc.load_gather(pan_i, [rows, rem_i])
                acc = acc + (ucol * icol) * w[f]
        z = acc + bias
        out_v[pl.ds(v * L, L)] = 1.0 / (1.0 + jnp.exp(-z))
        return carry

    lax.fori_loop(0, NWAVE, wave, 0)

    pltpu.sync_copy(out_v, out_hbm.at[pl.ds(base, BPW)])


_gmf = functools.partial(
    pl.kernel,
    out_type=jax.ShapeDtypeStruct((B,), jnp.float32),
    mesh=plsc.VectorSubcoreMesh(core_axis_name="c", subcore_axis_name="s"),
    scratch_types=[
        pltpu.VMEM((BPW,), jnp.int32),
        pltpu.VMEM((BPW,), jnp.int32),
        pltpu.VMEM((PROWS, TL), jnp.float32),
        pltpu.VMEM((PROWS, TL), jnp.float32),
        pltpu.VMEM((48,), jnp.float32),
        pltpu.VMEM((BPW,), jnp.float32),
        pltpu.SemaphoreType.DMA,
        pltpu.SemaphoreType.DMA,
    ],
    compiler_params=pltpu.CompilerParams(needs_layout_passes=False),
)(_gmf_body)


def kernel(users, items, user_table, item_table, pred_w, pred_b):
    params = jnp.concatenate([
        pred_w.reshape(-1).astype(jnp.float32),
        pred_b.reshape(-1).astype(jnp.float32),
        jnp.zeros((48 - F - 1,), jnp.float32),
    ])
    return _gmf(users.astype(jnp.int32), items.astype(jnp.int32),
                user_table.T, item_table.T, params)


# speedup vs baseline: 3.5479x; 1.0018x over previous
import functools

import jax
import jax.numpy as jnp
from jax import lax
from jax.experimental import pallas as pl
from jax.experimental.pallas import tpu as pltpu
from jax.experimental.pallas import tpu_sc as plsc

B = 16384
F = 32
NC = 2
NS = 16
NW = NC * NS
BPW = B // NW
L = 16
TS = 8
TL = 128
HALF = F // 2
NWAVE = BPW // L
PROWS = L * HALF


def _gmf_body(users_hbm, items_hbm, utab_hbm, itab_hbm, params_hbm, out_hbm,
              idx_u, idx_i, pan_u, pan_i, params_v, out_v, sem_u, sem_i):
    wid = lax.axis_index("s") * NC + lax.axis_index("c")
    base = wid * BPW

    pltpu.sync_copy(users_hbm.at[pl.ds(base, BPW)], idx_u)
    pltpu.sync_copy(items_hbm.at[pl.ds(base, BPW)], idx_i)
    pltpu.sync_copy(params_hbm, params_v)

    wv = [params_v[pl.ds(k * L, L)] for k in range(F // L)]
    bv = params_v[pl.ds(F // L * L, L)]
    w = [wv[f // L][f % L] for f in range(F)]
    bias = bv[0]
    lane = lax.iota(jnp.int32, L)

    def wave(v, carry):
        uvec = idx_u[pl.ds(v * L, L)]
        ivec = idx_i[pl.ds(v * L, L)]
        rem_u = uvec - (uvec // TL) * TL
        rem_i = ivec - (ivec // TL) * TL
        acc = jnp.zeros((L,), jnp.float32)
        for half in range(2):
            fr = half * HALF
            copies = []
            for k in range(L):
                qu = pl.multiple_of((uvec[k] // TL) * TL, TL)
                qi = pl.multiple_of((ivec[k] // TL) * TL, TL)
                copies.append(pltpu.async_copy(
                    utab_hbm.at[pl.ds(fr, HALF), pl.ds(qu, TL)],
                    pan_u.at[pl.ds(k * HALF, HALF)], sem_u))
                copies.append(pltpu.async_copy(
                    itab_hbm.at[pl.ds(fr, HALF), pl.ds(qi, TL)],
                    pan_i.at[pl.ds(k * HALF, HALF)], sem_i))
            for c in copies:
                c.wait()
            for fo in range(HALF):
                f = half * HALF + fo
                rows = lane * HALF + fo
                ucol = plsc.load_gather(pan_u, [rows, rem_u])
                icol = plsc.load_gather(pan_i, [rows, rem_i])
                acc = acc + (ucol * icol) * w[f]
        z = acc + bias
        out_v[pl.ds(v * L, L)] = 1.0 / (1.0 + jnp.exp(-z))
        return carry

    lax.fori_loop(0, NWAVE, wave, 0)

    pltpu.sync_copy(out_v, out_hbm.at[pl.ds(base, BPW)])


_gmf = functools.partial(
    pl.kernel,
    out_type=jax.ShapeDtypeStruct((B,), jnp.float32),
    mesh=plsc.VectorSubcoreMesh(core_axis_name="c", subcore_axis_name="s"),
    scratch_types=[
        pltpu.VMEM((BPW,), jnp.int32),
        pltpu.VMEM((BPW,), jnp.int32),
        pltpu.VMEM((PROWS, TL), jnp.float32),
        pltpu.VMEM((PROWS, TL), jnp.float32),
        pltpu.VMEM((48,), jnp.float32),
        pltpu.VMEM((BPW,), jnp.float32),
        pltpu.SemaphoreType.DMA,
        pltpu.SemaphoreType.DMA,
    ],
    compiler_params=pltpu.CompilerParams(needs_layout_passes=False),
)(_gmf_body)


def kernel(users, items, user_table, item_table, pred_w, pred_b):
    params = jnp.concatenate([
        pred_w.reshape(-1).astype(jnp.float32),
        pred_b.reshape(-1).astype(jnp.float32),
        jnp.zeros((48 - F - 1,), jnp.float32),
    ])
    return _gmf(users.astype(jnp.int32), items.astype(jnp.int32),
                user_table.T, item_table.T, params)
